# Initial kernel scaffold; baseline (speedup 1.0000x reference)
import jax
import jax.numpy as jnp
from jax.experimental import pallas as pl


def kernel(data_object, weights, N_i, m_i):
    raise NotImplementedError("write your pallas kernel here")


# speedup vs baseline: 1.2231x; 1.2231x over previous
import jax
import jax.numpy as jnp
from jax import lax
from jax.experimental import pallas as pl
from jax.experimental.pallas import tpu as pltpu

_N_Q = 8
_N_CODES = 1024
_D = 32


def _rqv_body(x_ref, w_ref, y_ref, idx_ref, sq_ref):
    x = x_ref[0]
    res = x
    acc = jnp.zeros_like(x)
    for i in range(_N_Q):
        w = w_ref[i]
        c2 = jnp.sum(w * w, axis=1, keepdims=True)
        r2 = jnp.sum(res * res, axis=0, keepdims=True)
        scores = lax.dot_general(
            w, res, (((1,), (0,)), ((), ())),
            preferred_element_type=jnp.float32,
            precision=lax.Precision.DEFAULT)
        obj = (r2 + c2) - 2.0 * scores
        mn = jnp.min(obj, axis=0, keepdims=True)
        iota = lax.broadcasted_iota(jnp.int32, obj.shape, 0)
        idx = jnp.min(jnp.where(obj == mn, iota, _N_CODES),
                      axis=0, keepdims=True)
        onehot = (iota == idx).astype(jnp.float32)
        q = lax.dot_general(
            w, onehot, (((0,), (0,)), ((), ())),
            preferred_element_type=jnp.float32,
            precision=lax.Precision.HIGHEST)
        acc = acc + q
        res = res - q
        idx_ref[0, pl.ds(i, 1), :] = idx
    y_ref[0] = acc
    diff = acc - x
    sq_ref[0] = jnp.full((1, 128), jnp.sum(diff * diff), dtype=jnp.float32)


def kernel(data_object, weights, N_i, m_i):
    b, d, s = data_object.shape
    grid = (b,)
    y, idx, sq = pl.pallas_call(
        _rqv_body,
        grid=grid,
        in_specs=[
            pl.BlockSpec((1, d, s), lambda i: (i, 0, 0)),
            pl.BlockSpec((_N_Q, _N_CODES, _D), lambda i: (0, 0, 0)),
        ],
        out_specs=[
            pl.BlockSpec((1, d, s), lambda i: (i, 0, 0)),
            pl.BlockSpec((1, _N_Q, s), lambda i: (i, 0, 0)),
            pl.BlockSpec((1, 1, 128), lambda i: (i, 0, 0)),
        ],
        out_shape=[
            jax.ShapeDtypeStruct((b, d, s), jnp.float32),
            jax.ShapeDtypeStruct((b, _N_Q, s), jnp.int32),
            jax.ShapeDtypeStruct((b, 1, 128), jnp.float32),
        ],
        compiler_params=pltpu.CompilerParams(
            dimension_semantics=("arbitrary",),
        ),
    )(data_object, weights)
    commitment_loss = jnp.sum(sq[:, 0, 0]) / (b * d * s)
    return y, commitment_loss, jnp.transpose(idx, (1, 0, 2))


# speedup vs baseline: 1.2236x; 1.0004x over previous
import jax
import jax.numpy as jnp
from jax import lax
from jax.experimental import pallas as pl
from jax.experimental.pallas import tpu as pltpu

_N_Q = 8
_N_CODES = 1024
_D = 32


def _rqv_body(x_ref, w_ref, y_ref, idx_ref, sq_ref):
    x = x_ref[0]
    res = x
    acc = jnp.zeros_like(x)
    for i in range(_N_Q):
        w = w_ref[i]
        c2 = jnp.sum(w * w, axis=1, keepdims=True)
        r2 = jnp.sum(res * res, axis=0, keepdims=True)
        scores = lax.dot_general(
            w, res, (((1,), (0,)), ((), ())),
            preferred_element_type=jnp.float32,
            precision=lax.Precision.DEFAULT)
        obj = (r2 + c2) - 2.0 * scores
        mn = jnp.min(obj, axis=0, keepdims=True)
        iota = lax.broadcasted_iota(jnp.int32, obj.shape, 0)
        idx = jnp.min(jnp.where(obj == mn, iota, _N_CODES),
                      axis=0, keepdims=True)
        onehot = (iota == idx).astype(jnp.float32)
        q = lax.dot_general(
            w, onehot, (((0,), (0,)), ((), ())),
            preferred_element_type=jnp.float32,
            precision=lax.Precision.HIGHEST)
        acc = acc + q
        res = res - q
        idx_ref[0, pl.ds(i, 1), :] = idx
    y_ref[0] = acc
    diff = acc - x
    sq_ref[0] = jnp.full((1, 128), jnp.sum(diff * diff), dtype=jnp.float32)


def kernel(data_object, weights, N_i, m_i):
    b, d, s = data_object.shape
    grid = (b,)
    y, idx, sq = pl.pallas_call(
        _rqv_body,
        grid=grid,
        in_specs=[
            pl.BlockSpec((1, d, s), lambda i: (i, 0, 0)),
            pl.BlockSpec((_N_Q, _N_CODES, _D), lambda i: (0, 0, 0)),
        ],
        out_specs=[
            pl.BlockSpec((1, d, s), lambda i: (i, 0, 0)),
            pl.BlockSpec((1, _N_Q, s), lambda i: (i, 0, 0)),
            pl.BlockSpec((1, 1, 128), lambda i: (i, 0, 0)),
        ],
        out_shape=[
            jax.ShapeDtypeStruct((b, d, s), jnp.float32),
            jax.ShapeDtypeStruct((b, _N_Q, s), jnp.int32),
            jax.ShapeDtypeStruct((b, 1, 128), jnp.float32),
        ],
        compiler_params=pltpu.CompilerParams(
            dimension_semantics=("parallel",),
        ),
    )(data_object, weights)
    commitment_loss = jnp.sum(sq[:, 0, 0]) / (b * d * s)
    return y, commitment_loss, jnp.transpose(idx, (1, 0, 2))
